# speedup vs baseline: 1.9984x; 1.9984x over previous
import functools

import jax
import jax.numpy as jnp
from jax import lax
from jax.experimental import pallas as pl
from jax.experimental.pallas import tpu as pltpu
from jax.experimental.pallas import tpu_sc as plsc

EMB = 1024
NUM_TOK = 4 * 4096

_info = plsc.get_sparse_core_info()
NC, NS = _info.num_cores, _info.num_subcores
NW = NC * NS

PER_W = NUM_TOK // NW
CHUNK = 64
NCHUNK = PER_W // CHUNK


def _body(pe_hbm, tok_hbm, out_hbm, idx_v, buf, gsem):
    wid = lax.axis_index("s") * NC + lax.axis_index("c")
    base = wid * PER_W

    pltpu.sync_copy(tok_hbm.at[wid], idx_v)

    @pl.loop(0, NCHUNK)
    def _chunk(t):
        pltpu.async_copy(pe_hbm.at[idx_v.at[t]], buf, gsem).wait()
        pltpu.sync_copy(buf, out_hbm.at[pl.ds(base + t * CHUNK, CHUNK)])


@jax.jit
def kernel(tok, pe):
    b, l = tok.shape
    tok3 = tok.reshape(NW, NCHUNK, CHUNK).astype(jnp.int32)
    out = pl.kernel(
        _body,
        out_type=jax.ShapeDtypeStruct((NUM_TOK, EMB), jnp.float32),
        mesh=plsc.VectorSubcoreMesh(core_axis_name="c", subcore_axis_name="s"),
        scratch_types=[
            pltpu.VMEM((NCHUNK, CHUNK), jnp.int32),
            pltpu.VMEM((CHUNK, EMB), jnp.float32),
            pltpu.SemaphoreType.DMA,
        ],
    )(pe, tok3)
    return out.reshape(b, l, EMB)


# speedup vs baseline: 2.0434x; 1.0225x over previous
import functools

import jax
import jax.numpy as jnp
from jax import lax
from jax.experimental import pallas as pl
from jax.experimental.pallas import tpu as pltpu
from jax.experimental.pallas import tpu_sc as plsc

EMB = 1024
NUM_TOK = 4 * 4096

_info = plsc.get_sparse_core_info()
NC, NS = _info.num_cores, _info.num_subcores
NW = NC * NS

PER_W = NUM_TOK // NW
CHUNK = 32
NCHUNK = PER_W // CHUNK
NBUF = 3
DEPTH = 2


def _body(pe_hbm, tok_hbm, out_hbm, idx_v, buf, gsem, ssem):
    wid = lax.axis_index("s") * NC + lax.axis_index("c")
    base = wid * PER_W

    pltpu.sync_copy(tok_hbm.at[wid], idx_v)

    gd = [None] * NCHUNK
    sd = [None] * NCHUNK

    def start_gather(t):
        b = t % NBUF
        gd[t] = pltpu.async_copy(pe_hbm.at[idx_v.at[t]], buf.at[b], gsem.at[b])

    def start_store(t):
        b = t % NBUF
        sd[t] = pltpu.async_copy(
            buf.at[b], out_hbm.at[pl.ds(base + t * CHUNK, CHUNK)], ssem.at[b]
        )

    for t in range(DEPTH):
        start_gather(t)
    for t in range(NCHUNK):
        gd[t].wait()
        start_store(t)
        tf = t + DEPTH
        if tf < NCHUNK:
            if tf >= NBUF:
                sd[tf - NBUF].wait()
            start_gather(tf)
    for t in range(NCHUNK - NBUF, NCHUNK):
        sd[t].wait()


@jax.jit
def kernel(tok, pe):
    b, l = tok.shape
    tok3 = tok.reshape(NW, NCHUNK, CHUNK).astype(jnp.int32)
    out = pl.kernel(
        _body,
        out_type=jax.ShapeDtypeStruct((NUM_TOK, EMB), jnp.float32),
        mesh=plsc.VectorSubcoreMesh(core_axis_name="c", subcore_axis_name="s"),
        scratch_types=[
            pltpu.VMEM((NCHUNK, CHUNK), jnp.int32),
            pltpu.VMEM((NBUF, CHUNK, EMB), jnp.float32),
            pltpu.SemaphoreType.DMA((NBUF,)),
            pltpu.SemaphoreType.DMA((NBUF,)),
        ],
    )(pe, tok3)
    return out.reshape(b, l, EMB)


# speedup vs baseline: 2.0567x; 1.0065x over previous
import jax
import jax.numpy as jnp
from jax import lax
from jax.experimental import pallas as pl
from jax.experimental.pallas import tpu as pltpu
from jax.experimental.pallas import tpu_sc as plsc

BATCH = 4
SEQ = 4096
EMB = 1024
NUM_TOK = BATCH * SEQ

_info = plsc.get_sparse_core_info()
NC, NS = _info.num_cores, _info.num_subcores
NW = NC * NS
W_PER_B = NW // BATCH

PER_W = NUM_TOK // NW
CHUNK = 16
NCHUNK = PER_W // CHUNK
NBUF = 7
DEPTH = 3


def _body(pe_hbm, tok_hbm, out_hbm, idx_v, buf, gsem, ssem):
    wid = lax.axis_index("s") * NC + lax.axis_index("c")
    b = wid // W_PER_B
    l_base = (wid % W_PER_B) * PER_W

    pltpu.sync_copy(tok_hbm.at[b, pl.ds(l_base, PER_W)], idx_v)

    gd = [None] * NCHUNK
    sd = [None] * NCHUNK

    def start_gather(t):
        r = t % NBUF
        gd[t] = pltpu.async_copy(
            pe_hbm.at[idx_v.at[pl.ds(t * CHUNK, CHUNK)]], buf.at[r], gsem.at[r]
        )

    def start_store(t):
        r = t % NBUF
        sd[t] = pltpu.async_copy(
            buf.at[r], out_hbm.at[b, pl.ds(l_base + t * CHUNK, CHUNK)], ssem.at[r]
        )

    for t in range(DEPTH):
        start_gather(t)
    for t in range(NCHUNK):
        gd[t].wait()
        start_store(t)
        tf = t + DEPTH
        if tf < NCHUNK:
            if tf >= NBUF:
                sd[tf - NBUF].wait()
            start_gather(tf)
    for t in range(NCHUNK - NBUF, NCHUNK):
        sd[t].wait()


@jax.jit
def kernel(tok, pe):
    return pl.kernel(
        _body,
        out_type=jax.ShapeDtypeStruct((BATCH, SEQ, EMB), jnp.float32),
        mesh=plsc.VectorSubcoreMesh(core_axis_name="c", subcore_axis_name="s"),
        scratch_types=[
            pltpu.VMEM((PER_W,), jnp.int32),
            pltpu.VMEM((NBUF, CHUNK, EMB), jnp.float32),
            pltpu.SemaphoreType.DMA((NBUF,)),
            pltpu.SemaphoreType.DMA((NBUF,)),
        ],
    )(pe, tok.astype(jnp.int32))


# speedup vs baseline: 2.1103x; 1.0261x over previous
import jax
import jax.numpy as jnp
from jax import lax
from jax.experimental import pallas as pl
from jax.experimental.pallas import tpu as pltpu
from jax.experimental.pallas import tpu_sc as plsc

BATCH = 4
SEQ = 4096
EMB = 1024
NUM_TOK = BATCH * SEQ

_info = plsc.get_sparse_core_info()
NC, NS = _info.num_cores, _info.num_subcores
NW = NC * NS
W_PER_B = NW // BATCH

PER_W = NUM_TOK // NW
CHUNK = 16
NCHUNK = PER_W // CHUNK
NBUF = 4
DEPTH = 2


def _body(pe_hbm, tok_hbm, out_hbm, idx_v, buf, gsem, ssem):
    wid = lax.axis_index("s") * NC + lax.axis_index("c")
    b = wid // W_PER_B
    l_base = (wid % W_PER_B) * PER_W

    pltpu.sync_copy(tok_hbm.at[b, pl.ds(l_base, PER_W)], idx_v)

    def _gather_desc(t, r):
        off = pl.multiple_of(t * CHUNK, 8)
        return pltpu.make_async_copy(
            pe_hbm.at[idx_v.at[pl.ds(off, CHUNK)]], buf.at[r], gsem.at[r]
        )

    def _store_desc(t, r):
        return pltpu.make_async_copy(
            buf.at[r], out_hbm.at[b, pl.ds(l_base + t * CHUNK, CHUNK)], ssem.at[r]
        )

    def start_gather(t, r):
        _gather_desc(t, r).start()

    def start_store(t, r):
        _store_desc(t, r).start()

    for t in range(DEPTH):
        start_gather(t, t % NBUF)

    @pl.loop(0, NCHUNK)
    def _step(t):
        tf = t + DEPTH
        rf = lax.rem(tf, NBUF)

        @pl.when(tf < NCHUNK)
        def _():
            @pl.when(tf >= NBUF)
            def _():
                _store_desc(0, rf).wait()

            start_gather(tf, rf)

        r = lax.rem(t, NBUF)
        _gather_desc(0, r).wait()
        start_store(t, r)

    for t in range(NCHUNK - NBUF, NCHUNK):
        _store_desc(0, t % NBUF).wait()


@jax.jit
def kernel(tok, pe):
    return pl.kernel(
        _body,
        out_type=jax.ShapeDtypeStruct((BATCH, SEQ, EMB), jnp.float32),
        mesh=plsc.VectorSubcoreMesh(core_axis_name="c", subcore_axis_name="s"),
        scratch_types=[
            pltpu.VMEM((PER_W,), jnp.int32),
            pltpu.VMEM((NBUF, CHUNK, EMB), jnp.float32),
            pltpu.SemaphoreType.DMA((NBUF,)),
            pltpu.SemaphoreType.DMA((NBUF,)),
        ],
    )(pe, tok.astype(jnp.int32))


# speedup vs baseline: 2.1177x; 1.0035x over previous
import jax
import jax.numpy as jnp
from jax import lax
from jax.experimental import pallas as pl
from jax.experimental.pallas import tpu as pltpu
from jax.experimental.pallas import tpu_sc as plsc

BATCH = 4
SEQ = 4096
EMB = 1024
NUM_TOK = BATCH * SEQ

_info = plsc.get_sparse_core_info()
NC, NS = _info.num_cores, _info.num_subcores
NW = NC * NS
W_PER_B = NW // BATCH

PER_W = NUM_TOK // NW
CHUNK = 32
NCHUNK = PER_W // CHUNK
NBUF = 3
DEPTH = 2


def _body(pe_hbm, tok_hbm, out_hbm, idx_v, buf, gsem, ssem):
    wid = lax.axis_index("s") * NC + lax.axis_index("c")
    b = wid // W_PER_B
    l_base = (wid % W_PER_B) * PER_W

    pltpu.sync_copy(tok_hbm.at[b, pl.ds(l_base, PER_W)], idx_v)

    def _gather_desc(t, r):
        off = pl.multiple_of(t * CHUNK, 8)
        return pltpu.make_async_copy(
            pe_hbm.at[idx_v.at[pl.ds(off, CHUNK)]], buf.at[r], gsem.at[r]
        )

    def _store_desc(t, r):
        return pltpu.make_async_copy(
            buf.at[r], out_hbm.at[b, pl.ds(l_base + t * CHUNK, CHUNK)], ssem.at[r]
        )

    def start_gather(t, r):
        _gather_desc(t, r).start()

    def start_store(t, r):
        _store_desc(t, r).start()

    for t in range(DEPTH):
        start_gather(t, t % NBUF)

    @pl.loop(0, NCHUNK)
    def _step(t):
        tf = t + DEPTH
        rf = lax.rem(tf, NBUF)

        @pl.when(tf < NCHUNK)
        def _():
            @pl.when(tf >= NBUF)
            def _():
                _store_desc(0, rf).wait()

            start_gather(tf, rf)

        r = lax.rem(t, NBUF)
        _gather_desc(0, r).wait()
        start_store(t, r)

    for t in range(NCHUNK - NBUF, NCHUNK):
        _store_desc(0, t % NBUF).wait()


@jax.jit
def kernel(tok, pe):
    return pl.kernel(
        _body,
        out_type=jax.ShapeDtypeStruct((BATCH, SEQ, EMB), jnp.float32),
        mesh=plsc.VectorSubcoreMesh(core_axis_name="c", subcore_axis_name="s"),
        scratch_types=[
            pltpu.VMEM((PER_W,), jnp.int32),
            pltpu.VMEM((NBUF, CHUNK, EMB), jnp.float32),
            pltpu.SemaphoreType.DMA((NBUF,)),
            pltpu.SemaphoreType.DMA((NBUF,)),
        ],
    )(pe, tok.astype(jnp.int32))


# speedup vs baseline: 2.1495x; 1.0150x over previous
import jax
import jax.numpy as jnp
from jax import lax
from jax.experimental import pallas as pl
from jax.experimental.pallas import tpu as pltpu
from jax.experimental.pallas import tpu_sc as plsc

BATCH = 4
SEQ = 4096
EMB = 1024
NUM_TOK = BATCH * SEQ

_info = plsc.get_sparse_core_info()
NC, NS = _info.num_cores, _info.num_subcores
NW = NC * NS
W_PER_B = NW // BATCH

PER_W = NUM_TOK // NW
CHUNK = 32
NCHUNK = PER_W // CHUNK
NBUF = 3
DEPTH = 2


def _body(pe_hbm, tok_hbm, out_hbm, idx_v, buf, gsem, ssem):
    wid = lax.axis_index("s") * NC + lax.axis_index("c")
    b = wid // W_PER_B
    l_base = (wid % W_PER_B) * PER_W

    PRE = 128
    pltpu.sync_copy(tok_hbm.at[b, pl.ds(l_base, PRE)], idx_v.at[pl.ds(0, PRE)])

    def _gather_desc(t, r):
        off = pl.multiple_of(t * CHUNK, 8)
        return pltpu.make_async_copy(
            pe_hbm.at[idx_v.at[pl.ds(off, CHUNK)]], buf.at[r], gsem.at[r]
        )

    def _store_desc(t, r):
        return pltpu.make_async_copy(
            buf.at[r], out_hbm.at[b, pl.ds(l_base + t * CHUNK, CHUNK)], ssem.at[r]
        )

    def start_gather(t, r):
        _gather_desc(t, r).start()

    def start_store(t, r):
        _store_desc(t, r).start()

    start_gather(0, 0)
    pltpu.sync_copy(
        tok_hbm.at[b, pl.ds(l_base + PRE, PER_W - PRE)],
        idx_v.at[pl.ds(PRE, PER_W - PRE)],
    )
    for t in range(1, DEPTH):
        start_gather(t, t % NBUF)

    @pl.loop(0, NCHUNK)
    def _step(t):
        tf = t + DEPTH
        rf = lax.rem(tf, NBUF)

        @pl.when(tf < NCHUNK)
        def _():
            @pl.when(tf >= NBUF)
            def _():
                _store_desc(0, rf).wait()

            start_gather(tf, rf)

        r = lax.rem(t, NBUF)
        _gather_desc(0, r).wait()
        start_store(t, r)

    for t in range(NCHUNK - NBUF, NCHUNK):
        _store_desc(0, t % NBUF).wait()


@jax.jit
def kernel(tok, pe):
    return pl.kernel(
        _body,
        out_type=jax.ShapeDtypeStruct((BATCH, SEQ, EMB), jnp.float32),
        mesh=plsc.VectorSubcoreMesh(core_axis_name="c", subcore_axis_name="s"),
        scratch_types=[
            pltpu.VMEM((PER_W,), jnp.int32),
            pltpu.VMEM((NBUF, CHUNK, EMB), jnp.float32),
            pltpu.SemaphoreType.DMA((NBUF,)),
            pltpu.SemaphoreType.DMA((NBUF,)),
        ],
    )(pe, tok.astype(jnp.int32))
